# baseline (speedup 1.0000x reference)
import functools

import jax
import jax.numpy as jnp
import numpy as np
from jax import lax
from jax.experimental import pallas as pl
from jax.experimental.pallas import tpu as pltpu
from jax.experimental.pallas import tpu_sc as plsc

G = 300
F = 128
N = G * F
E = 614400
D_TRI = F * (F - 1) // 2
C_BN = 1.0 / np.sqrt(1.0 + 1e-5).astype(np.float32)

NC = 2
NS = 16
NW = NC * NS

_TRI = np.full((F, F), D_TRI, np.int32)
_IU = np.triu_indices(F, k=1)
_TRI[_IU] = np.arange(D_TRI, dtype=np.int32)
_W1IDX = _TRI.reshape(F * F // 32, 32)


def _sc_mesh():
    return plsc.VectorSubcoreMesh(core_axis_name="c", subcore_axis_name="s")


_NCHUNK = 16


def _sc_prep_body(dst_r2, w1s, w1idx,
                  hist_out, w1full,
                  dbuf, histv, ib0, ib1, wb0, wb1,
                  gs0, gs1, ws0, ws1):
    c = lax.axis_index("c")
    s = lax.axis_index("s")
    w = c * NS + s

    idxb = (ib0, ib1)
    wbuf = (wb0, wb1)
    gsem = (gs0, gs1)
    wsem = (ws0, ws1)

    def _chunk(q):
        return w * _NCHUNK + q

    pltpu.sync_copy(w1idx.at[_chunk(0)], idxb[0])
    pltpu.async_copy(w1s.at[idxb[0]], wbuf[0], gsem[0])
    for q in range(_NCHUNK):
        cur, nxt = q % 2, (q + 1) % 2
        pltpu.make_async_copy(w1s.at[idxb[cur]], wbuf[cur], gsem[cur]).wait()
        pltpu.async_copy(wbuf[cur], w1full.at[pl.ds(_chunk(q) * 32, 32)],
                         wsem[cur])
        if q < _NCHUNK - 1:
            if q >= 1:
                pltpu.make_async_copy(
                    wbuf[nxt], w1full.at[pl.ds(_chunk(q - 1) * 32, 32)],
                    wsem[nxt]).wait()
            pltpu.sync_copy(w1idx.at[_chunk(q + 1)], idxb[nxt])
            pltpu.async_copy(w1s.at[idxb[nxt]], wbuf[nxt], gsem[nxt])

    zero16 = jnp.zeros((16,), jnp.int32)

    def _zero(i, _):
        histv[pl.ds(i * 16, 16)] = zero16
        return ()
    lax.fori_loop(0, N // 16, _zero, (), unroll=8)

    pltpu.sync_copy(dst_r2.at[w], dbuf)

    ones16 = jnp.ones((16,), jnp.int32)

    def _count(j, _):
        for k in range(8):
            v = dbuf[j, pl.ds(k * 16, 16)]
            cnt, last = plsc.scan_count(v)
            plsc.addupdate_scatter(histv, [v], cnt + ones16, mask=last)
        return ()
    lax.fori_loop(0, 150, _count, (), unroll=2)

    pltpu.sync_copy(histv, hist_out.at[w])

    pltpu.make_async_copy(
        wbuf[(_NCHUNK - 2) % 2],
        w1full.at[pl.ds(_chunk(_NCHUNK - 2) * 32, 32)],
        wsem[(_NCHUNK - 2) % 2]).wait()
    pltpu.make_async_copy(
        wbuf[(_NCHUNK - 1) % 2],
        w1full.at[pl.ds(_chunk(_NCHUNK - 1) * 32, 32)],
        wsem[(_NCHUNK - 1) % 2]).wait()


def _sc_prep(dst_r2, w1s, w1idx):
    return pl.kernel(
        _sc_prep_body,
        out_type=(
            jax.ShapeDtypeStruct((NW, N), jnp.int32),
            jax.ShapeDtypeStruct((F * F, 512), jnp.float32),
        ),
        mesh=_sc_mesh(),
        compiler_params=pltpu.CompilerParams(needs_layout_passes=False,
                                             use_tc_tiling_on_sc=False),
        scratch_types=[
            pltpu.VMEM((150, 128), jnp.int32),
            pltpu.VMEM((N,), jnp.int32),
            pltpu.VMEM((32,), jnp.int32),
            pltpu.VMEM((32,), jnp.int32),
            pltpu.VMEM((32, 512), jnp.float32),
            pltpu.VMEM((32, 512), jnp.float32),
            pltpu.SemaphoreType.DMA,
            pltpu.SemaphoreType.DMA,
            pltpu.SemaphoreType.DMA,
            pltpu.SemaphoreType.DMA,
        ],
    )(dst_r2, w1s, w1idx)


def _tc_w_body(w1_ref, bng_ref, bnb_ref, w1s_ref, bias_ref, bacc):
    k = pl.program_id(0)

    @pl.when(k == 0)
    def _():
        bacc[...] = jnp.zeros_like(bacc)

    rows = lax.broadcasted_iota(jnp.int32, (512, 1), 0) + k * 512
    valid = (rows < D_TRI).astype(jnp.float32)
    blk = w1_ref[...]
    w1s_ref[...] = blk * (bng_ref[...] * C_BN * valid)
    bacc[...] += jnp.sum(blk * (bnb_ref[...] * valid), axis=0, keepdims=True)

    @pl.when(k == pl.num_programs(0) - 1)
    def _():
        bias_ref[...] = bacc[...]


def _tc_w(w1a_part, bng_pad, bnb_pad):
    return pl.pallas_call(
        _tc_w_body,
        grid=(16,),
        in_specs=[
            pl.BlockSpec((512, 512), lambda k: (k, 0)),
            pl.BlockSpec((512, 1), lambda k: (k, 0)),
            pl.BlockSpec((512, 1), lambda k: (k, 0)),
        ],
        out_specs=[
            pl.BlockSpec((512, 512), lambda k: (k, 0)),
            pl.BlockSpec((1, 512), lambda k: (0, 0)),
        ],
        out_shape=[
            jax.ShapeDtypeStruct((8192, 512), jnp.float32),
            jax.ShapeDtypeStruct((1, 512), jnp.float32),
        ],
        scratch_shapes=[pltpu.VMEM((1, 512), jnp.float32)],
    )(w1a_part, bng_pad, bnb_pad)


_NBUF = 4
_NBATCH = 300


def _sc_scatter_body(ev_r, ytab, out,
                     ev, r0, r1, r2, r3, i0, i1, i2, i3, d0, d1, d2, d3,
                     zbuf, acc, sem0, sem1, sem2, sem3):
    rows = (r0, r1, r2, r3)
    idxs = (i0, i1, i2, i3)
    dsts = (d0, d1, d2, d3)
    sems = (sem0, sem1, sem2, sem3)

    c = lax.axis_index("c")
    s = lax.axis_index("s")

    zero16f = jnp.zeros((16,), jnp.float32)

    def _zerozb(j, _):
        zbuf[j, :] = zero16f
        return ()
    lax.fori_loop(0, 600, _zerozb, (), unroll=8)

    pltpu.sync_copy(ev_r.at[s], ev)

    for r in range(4):
        sl = 2 * r + c

        for q in range(4):
            pltpu.sync_copy(zbuf, acc.at[pl.ds(s * 2400 + q * 600, 600)])
        plsc.subcore_barrier()

        def _start(b, q):
            for k in range(8):
                v = ev[b, pl.ds(k * 16, 16)]
                src = lax.shift_right_logical(v, 16)
                idxs[q][pl.ds(k * 16, 16)] = src * 8 + sl
            pltpu.async_copy(ytab.at[idxs[q]], rows[q], sems[q])

        for p in range(_NBUF):
            _start(p, p)

        def _group(g, _):
            for q in range(_NBUF):
                b = g * _NBUF + q
                for k in range(8):
                    v = ev[b, pl.ds(k * 16, 16)]
                    dsts[q][pl.ds(k * 16, 16)] = v & 0xFFFF
                pltpu.make_async_copy(
                    ytab.at[idxs[q]], rows[q], sems[q]).wait()
                pltpu.sync_copy(rows[q], acc.at[dsts[q]], add=True)
                nb = b + _NBUF

                @pl.when(nb < _NBATCH)
                def _():
                    _start(nb, q)
            return ()
        lax.fori_loop(0, _NBATCH // _NBUF, _group, ())

        plsc.subcore_barrier()

        pltpu.sync_copy(acc.at[pl.ds(s * 2400, 2400)],
                        out.at[pl.ds(s * 2400, 2400), pl.ds(sl * 16, 16)])

        plsc.subcore_barrier()


def _sc_scatter(ev_r, ytab):
    return pl.kernel(
        _sc_scatter_body,
        out_type=jax.ShapeDtypeStruct((N, F), jnp.float32),
        mesh=_sc_mesh(),
        compiler_params=pltpu.CompilerParams(use_tc_tiling_on_sc=False),
        scratch_types=[
            pltpu.VMEM((_NBATCH, 128), jnp.int32),
            pltpu.VMEM((128, 16), jnp.float32),
            pltpu.VMEM((128, 16), jnp.float32),
            pltpu.VMEM((128, 16), jnp.float32),
            pltpu.VMEM((128, 16), jnp.float32),
            pltpu.VMEM((128,), jnp.int32),
            pltpu.VMEM((128,), jnp.int32),
            pltpu.VMEM((128,), jnp.int32),
            pltpu.VMEM((128,), jnp.int32),
            pltpu.VMEM((128,), jnp.int32),
            pltpu.VMEM((128,), jnp.int32),
            pltpu.VMEM((128,), jnp.int32),
            pltpu.VMEM((128,), jnp.int32),
            pltpu.VMEM((600, 16), jnp.float32),
            pltpu.VMEM_SHARED((N, 16), jnp.float32),
            pltpu.SemaphoreType.DMA,
            pltpu.SemaphoreType.DMA,
            pltpu.SemaphoreType.DMA,
            pltpu.SemaphoreType.DMA,
        ],
    )(ev_r, ytab)


def _tc_a_body(x_ref, w_ref, hist_ref, y_ref):
    deg = jnp.sum(hist_ref[...].astype(jnp.float32), axis=0) + 1.0
    dinv = lax.rsqrt(deg)
    xw = jnp.dot(x_ref[...], w_ref[...], preferred_element_type=jnp.float32)
    y_ref[...] = xw * dinv[:, None]


def _tc_a(x, w_gcn, hist):
    bn = 1280
    return pl.pallas_call(
        _tc_a_body,
        grid=(N // bn,),
        in_specs=[
            pl.BlockSpec((bn, F), lambda i: (i, 0)),
            pl.BlockSpec((F, F), lambda i: (0, 0)),
            pl.BlockSpec((NW, bn), lambda i: (0, i)),
        ],
        out_specs=pl.BlockSpec((bn, F), lambda i: (i, 0)),
        out_shape=jax.ShapeDtypeStruct((N, F), jnp.float32),
    )(x, w_gcn, hist)


def _tc_b_body(s_ref, y_ref, hist_ref,
               bg_ref, wa_ref, ba_ref, hm_ref):
    h = s_ref[...]
    deg = jnp.sum(hist_ref[...].astype(jnp.float32), axis=0) + 1.0
    dinv = lax.rsqrt(deg)
    h1 = jnp.tanh((h + y_ref[...]) * dinv[:, None] + bg_ref[...])
    logits = jnp.dot(h1, wa_ref[...], preferred_element_type=jnp.float32)
    logits = logits + ba_ref[...]
    m = jnp.max(logits, axis=1, keepdims=True)
    ex = jnp.exp(logits - m)
    aw = ex / jnp.sum(ex, axis=1, keepdims=True)
    hm_ref[...] = jnp.mean(h1 * aw, axis=0)[None, None, :]


def _tc_b(s, y, hist, b_gcn2, wa, ba2):
    return pl.pallas_call(
        _tc_b_body,
        grid=(G,),
        in_specs=[
            pl.BlockSpec((F, F), lambda i: (i, 0)),
            pl.BlockSpec((F, F), lambda i: (i, 0)),
            pl.BlockSpec((NW, F), lambda i: (0, i)),
            pl.BlockSpec((1, F), lambda i: (0, 0)),
            pl.BlockSpec((F, F), lambda i: (0, 0)),
            pl.BlockSpec((1, F), lambda i: (0, 0)),
        ],
        out_specs=pl.BlockSpec((1, 1, F), lambda i: (i, 0, 0)),
        out_shape=jax.ShapeDtypeStruct((G, 1, F), jnp.float32),
    )(s, y, hist, b_gcn2, wa, ba2)


def _tc_c_body(hm_ref, wg_ref, bg_ref, wat_ref, bat_ref, bnhg_ref, bnhb_ref,
               out_ref):
    y2 = jax.nn.relu(
        jnp.dot(hm_ref[...], wg_ref[...], preferred_element_type=jnp.float32)
        + bg_ref[...])
    l2 = jnp.dot(y2, wat_ref[...], preferred_element_type=jnp.float32)
    l2 = l2 + bat_ref[...]
    l2c = l2[:, 0:1]
    m = jnp.max(l2c, axis=0, keepdims=True)
    ex = jnp.exp(l2c - m)
    aw2 = ex / jnp.sum(ex, axis=0, keepdims=True)
    hg = jnp.sum(y2 * aw2, axis=0)
    out_ref[...] = (hg * C_BN * bnhg_ref[0, :] + bnhb_ref[0, :])[None, :]


def _tc_c(hm, wg, bg2, wat_pad, bat_pad, bnhg2, bnhb2):
    full = lambda shape: pl.BlockSpec(shape, lambda: tuple(0 for _ in shape))
    return pl.pallas_call(
        _tc_c_body,
        in_specs=[full((G, F)), full((F, F)), full((1, F)), full((F, F)),
                  full((1, F)), full((1, F)), full((1, F))],
        out_specs=full((1, F)),
        out_shape=jax.ShapeDtypeStruct((1, F), jnp.float32),
    )(hm, wg, bg2, wat_pad, bat_pad, bnhg2, bnhb2)


def _tc_d_body(xp_ref, w1f_ref, w1b_ref, hgbn_ref, bias1_ref,
               b1_ref, g1_ref, e1_ref, w2_ref, b2_ref, g2_ref, e2_ref,
               w3_ref, b3_ref, g3_ref, e3_ref, w4_ref, b4_ref,
               out_ref, acc):
    i = pl.program_id(0)

    @pl.when(i == 0)
    def _():
        acc[...] = jnp.zeros_like(acc)
        out_ref[...] = jnp.zeros_like(out_ref)

    acc[...] += jnp.dot(xp_ref[...], w1f_ref[...],
                        preferred_element_type=jnp.float32)

    @pl.when(i == pl.num_programs(0) - 1)
    def _():
        hg_term = jnp.dot(hgbn_ref[...], w1b_ref[...],
                          preferred_element_type=jnp.float32)
        z1 = acc[...] + bias1_ref[...] + hg_term + b1_ref[...]
        z1 = jax.nn.relu(z1 * (C_BN * g1_ref[...]) + e1_ref[...])
        z2 = jnp.dot(z1, w2_ref[...], preferred_element_type=jnp.float32)
        z2 = jax.nn.relu((z2 + b2_ref[...]) * (C_BN * g2_ref[...])
                         + e2_ref[...])
        z3 = jnp.dot(z2, w3_ref[...], preferred_element_type=jnp.float32)
        z3 = jax.nn.relu((z3 + b3_ref[...]) * (C_BN * g3_ref[...])
                         + e3_ref[...])
        z4 = jnp.dot(z3, w4_ref[...], preferred_element_type=jnp.float32)
        out_ref[...] = z4 + b4_ref[...]


def _tc_d(xp, w1full, w1b, hgbn, bias1, b1, g1, e1, w2, b2, g2, e2,
          w3, b3, g3, e3, w4p, b4p):
    const = lambda shape: pl.BlockSpec(shape, lambda i: tuple(0 for _ in shape))
    return pl.pallas_call(
        _tc_d_body,
        grid=(F,),
        in_specs=[
            pl.BlockSpec((G, F), lambda i: (0, i)),
            pl.BlockSpec((F, 512), lambda i: (i, 0)),
            const((F, 512)),
            const((1, F)),
            const((1, 512)),
            const((1, 512)), const((1, 512)), const((1, 512)),
            const((512, 256)),
            const((1, 256)), const((1, 256)), const((1, 256)),
            const((256, 256)),
            const((1, 256)), const((1, 256)), const((1, 256)),
            const((256, F)),
            const((1, F)),
        ],
        out_specs=const((G, F)),
        out_shape=jax.ShapeDtypeStruct((G, F), jnp.float32),
        scratch_shapes=[
            pltpu.VMEM((G, 512), jnp.float32),
        ],
    )(xp, w1full, w1b, hgbn, bias1, b1, g1, e1, w2, b2, g2, e2,
      w3, b3, g3, e3, w4p, b4p)


def kernel(x, edge_index, batch, W_gcn, b_gcn, Wa, ba, Wg, bg, Wat, bat,
           bn_g, bn_b, bnh_g, bnh_b, W1, b1, g1, e1, W2, b2, g2, e2,
           W3, b3, g3, e3, W4, b4):
    dst_r2 = edge_index[1].reshape(NW, 150, 128)
    ev_r = ((edge_index[0] << 16) | edge_index[1]).reshape(NS, _NBATCH, 128)

    w1idx = jnp.asarray(_W1IDX)
    w1s, bias1 = _tc_w(W1[:8192],
                       jnp.pad(bn_g, (0, 64)).reshape(8192, 1),
                       jnp.pad(bn_b, (0, 64)).reshape(8192, 1))

    hist, w1full = _sc_prep(dst_r2, w1s, w1idx)

    y = _tc_a(x, W_gcn, hist)
    ytab = y.reshape(N * 8, 16)

    s = _sc_scatter(ev_r, ytab)

    hm = _tc_b(s, y, hist, b_gcn.reshape(1, F), Wa, ba.reshape(1, F))
    hm2 = hm.reshape(G, F)

    hgbn = _tc_c(hm2, Wg, bg.reshape(1, F),
                 jnp.pad(Wat, ((0, 0), (0, F - 1))),
                 jnp.pad(bat, (0, F - 1)).reshape(1, F),
                 bnh_g.reshape(1, F), bnh_b.reshape(1, F))

    xp = x.reshape(G, F * F)
    zd = _tc_d(xp, w1full, W1[D_TRI:], hgbn, bias1,
               b1.reshape(1, 512), g1.reshape(1, 512), e1.reshape(1, 512),
               W2, b2.reshape(1, 256), g2.reshape(1, 256), e2.reshape(1, 256),
               W3, b3.reshape(1, 256), g3.reshape(1, 256), e3.reshape(1, 256),
               jnp.pad(W4, ((0, 0), (0, F - 2))),
               jnp.pad(b4, (0, F - 2)).reshape(1, F))
    return zd[:, :2]


# speedup vs baseline: 1.4266x; 1.4266x over previous
import functools

import jax
import jax.numpy as jnp
import numpy as np
from jax import lax
from jax.experimental import pallas as pl
from jax.experimental.pallas import tpu as pltpu
from jax.experimental.pallas import tpu_sc as plsc

G = 300
F = 128
N = G * F
E = 614400
D_TRI = F * (F - 1) // 2
C_BN = 1.0 / np.sqrt(1.0 + 1e-5).astype(np.float32)

NC = 2
NS = 16
NW = NC * NS

_TRI = D_TRI + np.arange(F * F, dtype=np.int32).reshape(F, F) % 64
_IU = np.triu_indices(F, k=1)
_TRI[_IU] = np.arange(D_TRI, dtype=np.int32)
_W1IDX = _TRI.reshape(F * F // 32, 32)


def _sc_mesh():
    return plsc.VectorSubcoreMesh(core_axis_name="c", subcore_axis_name="s")


_NCHUNK = 16


def _sc_prep_body(dst_r2, w1s, w1idx,
                  hist_out, w1full,
                  dbuf, histv, ib0, ib1, wb0, wb1,
                  gs0, gs1, ws0, ws1):
    c = lax.axis_index("c")
    s = lax.axis_index("s")
    w = c * NS + s

    idxb = (ib0, ib1)
    wbuf = (wb0, wb1)
    gsem = (gs0, gs1)
    wsem = (ws0, ws1)

    def _chunk(q):
        return w * _NCHUNK + q

    pltpu.sync_copy(w1idx.at[_chunk(0)], idxb[0])
    pltpu.async_copy(w1s.at[idxb[0]], wbuf[0], gsem[0])
    for q in range(_NCHUNK):
        cur, nxt = q % 2, (q + 1) % 2
        pltpu.make_async_copy(w1s.at[idxb[cur]], wbuf[cur], gsem[cur]).wait()
        pltpu.async_copy(wbuf[cur], w1full.at[pl.ds(_chunk(q) * 32, 32)],
                         wsem[cur])
        if q < _NCHUNK - 1:
            if q >= 1:
                pltpu.make_async_copy(
                    wbuf[nxt], w1full.at[pl.ds(_chunk(q - 1) * 32, 32)],
                    wsem[nxt]).wait()
            pltpu.sync_copy(w1idx.at[_chunk(q + 1)], idxb[nxt])
            pltpu.async_copy(w1s.at[idxb[nxt]], wbuf[nxt], gsem[nxt])

    zero16 = jnp.zeros((16,), jnp.int32)

    def _zero(i, _):
        histv[pl.ds(i * 16, 16)] = zero16
        return ()
    lax.fori_loop(0, N // 16, _zero, (), unroll=8)

    pltpu.sync_copy(dst_r2.at[w], dbuf)

    ones16 = jnp.ones((16,), jnp.int32)

    def _count(j, _):
        for k in range(8):
            v = dbuf[j, pl.ds(k * 16, 16)]
            cnt, last = plsc.scan_count(v)
            plsc.addupdate_scatter(histv, [v], cnt + ones16, mask=last)
        return ()
    lax.fori_loop(0, 150, _count, (), unroll=2)

    pltpu.sync_copy(histv, hist_out.at[w])

    pltpu.make_async_copy(
        wbuf[(_NCHUNK - 2) % 2],
        w1full.at[pl.ds(_chunk(_NCHUNK - 2) * 32, 32)],
        wsem[(_NCHUNK - 2) % 2]).wait()
    pltpu.make_async_copy(
        wbuf[(_NCHUNK - 1) % 2],
        w1full.at[pl.ds(_chunk(_NCHUNK - 1) * 32, 32)],
        wsem[(_NCHUNK - 1) % 2]).wait()


def _sc_prep(dst_r2, w1s, w1idx):
    return pl.kernel(
        _sc_prep_body,
        out_type=(
            jax.ShapeDtypeStruct((NW, N), jnp.int32),
            jax.ShapeDtypeStruct((F * F, 512), jnp.float32),
        ),
        mesh=_sc_mesh(),
        compiler_params=pltpu.CompilerParams(needs_layout_passes=False,
                                             use_tc_tiling_on_sc=False),
        scratch_types=[
            pltpu.VMEM((150, 128), jnp.int32),
            pltpu.VMEM((N,), jnp.int32),
            pltpu.VMEM((32,), jnp.int32),
            pltpu.VMEM((32,), jnp.int32),
            pltpu.VMEM((32, 512), jnp.float32),
            pltpu.VMEM((32, 512), jnp.float32),
            pltpu.SemaphoreType.DMA,
            pltpu.SemaphoreType.DMA,
            pltpu.SemaphoreType.DMA,
            pltpu.SemaphoreType.DMA,
        ],
    )(dst_r2, w1s, w1idx)


def _tc_w_body(w1_ref, bng_ref, bnb_ref, w1s_ref, bias_ref, bacc):
    k = pl.program_id(0)

    @pl.when(k == 0)
    def _():
        bacc[...] = jnp.zeros_like(bacc)

    rows = lax.broadcasted_iota(jnp.int32, (512, 1), 0) + k * 512
    valid = (rows < D_TRI).astype(jnp.float32)
    blk = w1_ref[...]
    w1s_ref[...] = blk * (bng_ref[...] * C_BN * valid)
    bacc[...] += jnp.sum(blk * (bnb_ref[...] * valid), axis=0, keepdims=True)

    @pl.when(k == pl.num_programs(0) - 1)
    def _():
        bias_ref[...] = bacc[...]


def _tc_w(w1a_part, bng_pad, bnb_pad):
    return pl.pallas_call(
        _tc_w_body,
        grid=(16,),
        in_specs=[
            pl.BlockSpec((512, 512), lambda k: (k, 0)),
            pl.BlockSpec((512, 1), lambda k: (k, 0)),
            pl.BlockSpec((512, 1), lambda k: (k, 0)),
        ],
        out_specs=[
            pl.BlockSpec((512, 512), lambda k: (k, 0)),
            pl.BlockSpec((1, 512), lambda k: (0, 0)),
        ],
        out_shape=[
            jax.ShapeDtypeStruct((8192, 512), jnp.float32),
            jax.ShapeDtypeStruct((1, 512), jnp.float32),
        ],
        scratch_shapes=[pltpu.VMEM((1, 512), jnp.float32)],
    )(w1a_part, bng_pad, bnb_pad)


_NBUF = 4
_NBATCH = 300


def _sc_scatter_body(ev_r, ytab, out,
                     ev, r0, r1, r2, r3, i0, i1, i2, i3, d0, d1, d2, d3,
                     zbuf, acc, sem0, sem1, sem2, sem3,
                     ssem0, ssem1, ssem2, ssem3):
    rows = (r0, r1, r2, r3)
    idxs = (i0, i1, i2, i3)
    dsts = (d0, d1, d2, d3)
    sems = (sem0, sem1, sem2, sem3)
    ssems = (ssem0, ssem1, ssem2, ssem3)

    c = lax.axis_index("c")
    s = lax.axis_index("s")

    zero16f = jnp.zeros((16,), jnp.float32)

    def _zerozb(j, _):
        zbuf[j, :] = zero16f
        return ()
    lax.fori_loop(0, 600, _zerozb, (), unroll=8)

    pltpu.sync_copy(ev_r.at[s], ev)

    for r in range(4):
        sl = 2 * r + c

        for q in range(4):
            pltpu.sync_copy(zbuf, acc.at[pl.ds(s * 2400 + q * 600, 600)])
        plsc.subcore_barrier()

        def _start(b, q):
            for k in range(8):
                v = ev[b, pl.ds(k * 16, 16)]
                src = lax.shift_right_logical(v, 16)
                idxs[q][pl.ds(k * 16, 16)] = src * 8 + sl
            pltpu.async_copy(ytab.at[idxs[q]], rows[q], sems[q])

        _start(0, 0)
        _start(1, 1)

        def _group(g, _):
            for q in range(_NBUF):
                b = g * _NBUF + q
                pltpu.make_async_copy(
                    ytab.at[idxs[q]], rows[q], sems[q]).wait()
                for k in range(8):
                    v = ev[b, pl.ds(k * 16, 16)]
                    dsts[q][pl.ds(k * 16, 16)] = v & 0xFFFF
                pltpu.make_async_copy(
                    rows[q], acc.at[dsts[q]], ssems[q]).start(add=True)
                bf = b + 2
                qf = (q + 2) % _NBUF

                @pl.when(bf < _NBATCH)
                def _():
                    @pl.when(bf >= _NBUF)
                    def _():
                        pltpu.make_async_copy(
                            rows[qf], acc.at[dsts[qf]], ssems[qf]).wait()
                    _start(bf, qf)
            return ()
        lax.fori_loop(0, _NBATCH // _NBUF, _group, ())

        for q in range(_NBUF):
            pltpu.make_async_copy(rows[q], acc.at[dsts[q]], ssems[q]).wait()

        plsc.subcore_barrier()

        pltpu.sync_copy(acc.at[pl.ds(s * 2400, 2400)],
                        out.at[pl.ds(s * 2400, 2400), pl.ds(sl * 16, 16)])

        plsc.subcore_barrier()


def _sc_scatter(ev_r, ytab):
    return pl.kernel(
        _sc_scatter_body,
        out_type=jax.ShapeDtypeStruct((N, F), jnp.float32),
        mesh=_sc_mesh(),
        compiler_params=pltpu.CompilerParams(use_tc_tiling_on_sc=False),
        scratch_types=[
            pltpu.VMEM((_NBATCH, 128), jnp.int32),
            pltpu.VMEM((128, 16), jnp.float32),
            pltpu.VMEM((128, 16), jnp.float32),
            pltpu.VMEM((128, 16), jnp.float32),
            pltpu.VMEM((128, 16), jnp.float32),
            pltpu.VMEM((128,), jnp.int32),
            pltpu.VMEM((128,), jnp.int32),
            pltpu.VMEM((128,), jnp.int32),
            pltpu.VMEM((128,), jnp.int32),
            pltpu.VMEM((128,), jnp.int32),
            pltpu.VMEM((128,), jnp.int32),
            pltpu.VMEM((128,), jnp.int32),
            pltpu.VMEM((128,), jnp.int32),
            pltpu.VMEM((600, 16), jnp.float32),
            pltpu.VMEM_SHARED((N, 16), jnp.float32),
            pltpu.SemaphoreType.DMA,
            pltpu.SemaphoreType.DMA,
            pltpu.SemaphoreType.DMA,
            pltpu.SemaphoreType.DMA,
            pltpu.SemaphoreType.DMA,
            pltpu.SemaphoreType.DMA,
            pltpu.SemaphoreType.DMA,
            pltpu.SemaphoreType.DMA,
        ],
    )(ev_r, ytab)


def _tc_a_body(x_ref, w_ref, hist_ref, y_ref):
    deg = jnp.sum(hist_ref[...].astype(jnp.float32), axis=0) + 1.0
    dinv = lax.rsqrt(deg)
    xw = jnp.dot(x_ref[...], w_ref[...], preferred_element_type=jnp.float32)
    y_ref[...] = xw * dinv[:, None]


def _tc_a(x, w_gcn, hist):
    bn = 1280
    return pl.pallas_call(
        _tc_a_body,
        grid=(N // bn,),
        in_specs=[
            pl.BlockSpec((bn, F), lambda i: (i, 0)),
            pl.BlockSpec((F, F), lambda i: (0, 0)),
            pl.BlockSpec((NW, bn), lambda i: (0, i)),
        ],
        out_specs=pl.BlockSpec((bn, F), lambda i: (i, 0)),
        out_shape=jax.ShapeDtypeStruct((N, F), jnp.float32),
    )(x, w_gcn, hist)


_GB = 10


def _tc_b_body(s_ref, y_ref, hist_ref,
               bg_ref, wa_ref, ba_ref, hm_ref):
    h = s_ref[...]
    deg = jnp.sum(hist_ref[...].astype(jnp.float32), axis=0) + 1.0
    dinv = lax.rsqrt(deg)
    h1 = jnp.tanh((h + y_ref[...]) * dinv[:, None] + bg_ref[...])
    logits = jnp.dot(h1, wa_ref[...], preferred_element_type=jnp.float32)
    logits = logits + ba_ref[...]
    m = jnp.max(logits, axis=1, keepdims=True)
    ex = jnp.exp(logits - m)
    aw = ex / jnp.sum(ex, axis=1, keepdims=True)
    ga = h1 * aw
    hm_ref[...] = jnp.mean(ga.reshape(_GB, F, F), axis=1)[:, None, :]


def _tc_b(s, y, hist, b_gcn2, wa, ba2):
    return pl.pallas_call(
        _tc_b_body,
        grid=(G // _GB,),
        in_specs=[
            pl.BlockSpec((_GB * F, F), lambda i: (i, 0)),
            pl.BlockSpec((_GB * F, F), lambda i: (i, 0)),
            pl.BlockSpec((NW, _GB * F), lambda i: (0, i)),
            pl.BlockSpec((1, F), lambda i: (0, 0)),
            pl.BlockSpec((F, F), lambda i: (0, 0)),
            pl.BlockSpec((1, F), lambda i: (0, 0)),
        ],
        out_specs=pl.BlockSpec((_GB, 1, F), lambda i: (i, 0, 0)),
        out_shape=jax.ShapeDtypeStruct((G, 1, F), jnp.float32),
    )(s, y, hist, b_gcn2, wa, ba2)


def _tc_c_body(hm_ref, wg_ref, bg_ref, wat_ref, bat_ref, bnhg_ref, bnhb_ref,
               out_ref):
    y2 = jax.nn.relu(
        jnp.dot(hm_ref[...], wg_ref[...], preferred_element_type=jnp.float32)
        + bg_ref[...])
    l2 = jnp.dot(y2, wat_ref[...], preferred_element_type=jnp.float32)
    l2 = l2 + bat_ref[...]
    l2c = l2[:, 0:1]
    m = jnp.max(l2c, axis=0, keepdims=True)
    ex = jnp.exp(l2c - m)
    aw2 = ex / jnp.sum(ex, axis=0, keepdims=True)
    hg = jnp.sum(y2 * aw2, axis=0)
    out_ref[...] = (hg * C_BN * bnhg_ref[0, :] + bnhb_ref[0, :])[None, :]


def _tc_c(hm, wg, bg2, wat_pad, bat_pad, bnhg2, bnhb2):
    full = lambda shape: pl.BlockSpec(shape, lambda: tuple(0 for _ in shape))
    return pl.pallas_call(
        _tc_c_body,
        in_specs=[full((G, F)), full((F, F)), full((1, F)), full((F, F)),
                  full((1, F)), full((1, F)), full((1, F))],
        out_specs=full((1, F)),
        out_shape=jax.ShapeDtypeStruct((1, F), jnp.float32),
    )(hm, wg, bg2, wat_pad, bat_pad, bnhg2, bnhb2)


def _tc_d_body(xp_ref, w1f_ref, w1b_ref, hgbn_ref, bias1_ref,
               b1_ref, g1_ref, e1_ref, w2_ref, b2_ref, g2_ref, e2_ref,
               w3_ref, b3_ref, g3_ref, e3_ref, w4_ref, b4_ref,
               out_ref, acc):
    i = pl.program_id(0)

    @pl.when(i == 0)
    def _():
        acc[...] = jnp.zeros_like(acc)
        out_ref[...] = jnp.zeros_like(out_ref)

    acc[...] += jnp.dot(xp_ref[...], w1f_ref[...],
                        preferred_element_type=jnp.float32)

    @pl.when(i == pl.num_programs(0) - 1)
    def _():
        hg_term = jnp.dot(hgbn_ref[...], w1b_ref[...],
                          preferred_element_type=jnp.float32)
        z1 = acc[...] + bias1_ref[...] + hg_term + b1_ref[...]
        z1 = jax.nn.relu(z1 * (C_BN * g1_ref[...]) + e1_ref[...])
        z2 = jnp.dot(z1, w2_ref[...], preferred_element_type=jnp.float32)
        z2 = jax.nn.relu((z2 + b2_ref[...]) * (C_BN * g2_ref[...])
                         + e2_ref[...])
        z3 = jnp.dot(z2, w3_ref[...], preferred_element_type=jnp.float32)
        z3 = jax.nn.relu((z3 + b3_ref[...]) * (C_BN * g3_ref[...])
                         + e3_ref[...])
        z4 = jnp.dot(z3, w4_ref[...], preferred_element_type=jnp.float32)
        out_ref[...] = z4 + b4_ref[...]


def _tc_d(xp, w1full, w1b, hgbn, bias1, b1, g1, e1, w2, b2, g2, e2,
          w3, b3, g3, e3, w4p, b4p):
    const = lambda shape: pl.BlockSpec(shape, lambda i: tuple(0 for _ in shape))
    return pl.pallas_call(
        _tc_d_body,
        grid=(F * F // 512,),
        in_specs=[
            pl.BlockSpec((G, 512), lambda i: (0, i)),
            pl.BlockSpec((512, 512), lambda i: (i, 0)),
            const((F, 512)),
            const((1, F)),
            const((1, 512)),
            const((1, 512)), const((1, 512)), const((1, 512)),
            const((512, 256)),
            const((1, 256)), const((1, 256)), const((1, 256)),
            const((256, 256)),
            const((1, 256)), const((1, 256)), const((1, 256)),
            const((256, F)),
            const((1, F)),
        ],
        out_specs=const((G, F)),
        out_shape=jax.ShapeDtypeStruct((G, F), jnp.float32),
        scratch_shapes=[
            pltpu.VMEM((G, 512), jnp.float32),
        ],
    )(xp, w1full, w1b, hgbn, bias1, b1, g1, e1, w2, b2, g2, e2,
      w3, b3, g3, e3, w4p, b4p)


def kernel(x, edge_index, batch, W_gcn, b_gcn, Wa, ba, Wg, bg, Wat, bat,
           bn_g, bn_b, bnh_g, bnh_b, W1, b1, g1, e1, W2, b2, g2, e2,
           W3, b3, g3, e3, W4, b4):
    dst_r2 = edge_index[1].reshape(NW, 150, 128)
    ev_r = ((edge_index[0] << 16) | edge_index[1]).reshape(NS, _NBATCH, 128)

    w1idx = jnp.asarray(_W1IDX)
    w1s, bias1 = _tc_w(W1[:8192],
                       jnp.pad(bn_g, (0, 64)).reshape(8192, 1),
                       jnp.pad(bn_b, (0, 64)).reshape(8192, 1))

    hist, w1full = _sc_prep(dst_r2, w1s, w1idx)

    y = _tc_a(x, W_gcn, hist)
    ytab = y.reshape(N * 8, 16)

    s = _sc_scatter(ev_r, ytab)

    hm = _tc_b(s, y, hist, b_gcn.reshape(1, F), Wa, ba.reshape(1, F))
    hm2 = hm.reshape(G, F)

    hgbn = _tc_c(hm2, Wg, bg.reshape(1, F),
                 jnp.pad(Wat, ((0, 0), (0, F - 1))),
                 jnp.pad(bat, (0, F - 1)).reshape(1, F),
                 bnh_g.reshape(1, F), bnh_b.reshape(1, F))

    xp = x.reshape(G, F * F)
    zd = _tc_d(xp, w1full, W1[D_TRI:], hgbn, bias1,
               b1.reshape(1, 512), g1.reshape(1, 512), e1.reshape(1, 512),
               W2, b2.reshape(1, 256), g2.reshape(1, 256), e2.reshape(1, 256),
               W3, b3.reshape(1, 256), g3.reshape(1, 256), e3.reshape(1, 256),
               jnp.pad(W4, ((0, 0), (0, F - 2))),
               jnp.pad(b4, (0, F - 2)).reshape(1, F))
    return zd[:, :2]


# speedup vs baseline: 2.0957x; 1.4690x over previous
import functools

import jax
import jax.numpy as jnp
import numpy as np
from jax import lax
from jax.experimental import pallas as pl
from jax.experimental.pallas import tpu as pltpu
from jax.experimental.pallas import tpu_sc as plsc

G = 300
F = 128
N = G * F
E = 614400
D_TRI = F * (F - 1) // 2
C_BN = 1.0 / np.sqrt(1.0 + 1e-5).astype(np.float32)

NC = 2
NS = 16
NW = NC * NS

_TRI = D_TRI + np.arange(F * F, dtype=np.int32).reshape(F, F) % 64
_IU = np.triu_indices(F, k=1)
_TRI[_IU] = np.arange(D_TRI, dtype=np.int32)
_W1IDX = _TRI.reshape(F * F // 32, 32)


def _sc_mesh():
    return plsc.VectorSubcoreMesh(core_axis_name="c", subcore_axis_name="s")


def _sc_hist_body(dst_r2, hist_out, dbuf, histv):
    c = lax.axis_index("c")
    s = lax.axis_index("s")
    w = c * NS + s

    zero16 = jnp.zeros((16,), jnp.int32)

    def _zero(i, _):
        histv[pl.ds(i * 16, 16)] = zero16
        return ()
    lax.fori_loop(0, N // 16, _zero, (), unroll=8)

    pltpu.sync_copy(dst_r2.at[w], dbuf)

    ones16 = jnp.ones((16,), jnp.int32)

    def _count(j, _):
        for k in range(8):
            v = dbuf[j, pl.ds(k * 16, 16)]
            cnt, last = plsc.scan_count(v)
            plsc.addupdate_scatter(histv, [v], cnt + ones16, mask=last)
        return ()
    lax.fori_loop(0, 150, _count, (), unroll=2)

    pltpu.sync_copy(histv, hist_out.at[w])


def _sc_hist(dst_r2):
    return pl.kernel(
        _sc_hist_body,
        out_type=jax.ShapeDtypeStruct((NW, N), jnp.int32),
        mesh=_sc_mesh(),
        compiler_params=pltpu.CompilerParams(needs_layout_passes=False,
                                             use_tc_tiling_on_sc=False),
        scratch_types=[
            pltpu.VMEM((150, 128), jnp.int32),
            pltpu.VMEM((N,), jnp.int32),
        ],
    )(dst_r2)


_NCHUNK = 16


def _sc_w1x_body(w1s, w1idx, w1full,
                 ib0, ib1, wb0, wb1, gs0, gs1, ws0, ws1):
    c = lax.axis_index("c")
    s = lax.axis_index("s")
    w = c * NS + s

    idxb = (ib0, ib1)
    wbuf = (wb0, wb1)
    gsem = (gs0, gs1)
    wsem = (ws0, ws1)

    def _chunk(q):
        return w * _NCHUNK + q

    pltpu.sync_copy(w1idx.at[_chunk(0)], idxb[0])
    pltpu.async_copy(w1s.at[idxb[0]], wbuf[0], gsem[0])
    for q in range(_NCHUNK):
        cur, nxt = q % 2, (q + 1) % 2
        pltpu.make_async_copy(w1s.at[idxb[cur]], wbuf[cur], gsem[cur]).wait()
        pltpu.async_copy(wbuf[cur], w1full.at[pl.ds(_chunk(q) * 32, 32)],
                         wsem[cur])
        if q < _NCHUNK - 1:
            if q >= 1:
                pltpu.make_async_copy(
                    wbuf[nxt], w1full.at[pl.ds(_chunk(q - 1) * 32, 32)],
                    wsem[nxt]).wait()
            pltpu.sync_copy(w1idx.at[_chunk(q + 1)], idxb[nxt])
            pltpu.async_copy(w1s.at[idxb[nxt]], wbuf[nxt], gsem[nxt])

    pltpu.make_async_copy(
        wbuf[(_NCHUNK - 2) % 2],
        w1full.at[pl.ds(_chunk(_NCHUNK - 2) * 32, 32)],
        wsem[(_NCHUNK - 2) % 2]).wait()
    pltpu.make_async_copy(
        wbuf[(_NCHUNK - 1) % 2],
        w1full.at[pl.ds(_chunk(_NCHUNK - 1) * 32, 32)],
        wsem[(_NCHUNK - 1) % 2]).wait()


def _sc_w1x(w1s, w1idx):
    return pl.kernel(
        _sc_w1x_body,
        out_type=jax.ShapeDtypeStruct((F * F, 512), jnp.float32),
        mesh=_sc_mesh(),
        scratch_types=[
            pltpu.VMEM((32,), jnp.int32),
            pltpu.VMEM((32,), jnp.int32),
            pltpu.VMEM((32, 512), jnp.float32),
            pltpu.VMEM((32, 512), jnp.float32),
            pltpu.SemaphoreType.DMA,
            pltpu.SemaphoreType.DMA,
            pltpu.SemaphoreType.DMA,
            pltpu.SemaphoreType.DMA,
        ],
    )(w1s, w1idx)


def _tc_w_body(w1_ref, bng_ref, bnb_ref, w1s_ref, bias_ref, bacc):
    k = pl.program_id(0)

    @pl.when(k == 0)
    def _():
        bacc[...] = jnp.zeros_like(bacc)

    rows = lax.broadcasted_iota(jnp.int32, (512, 1), 0) + k * 512
    valid = (rows < D_TRI).astype(jnp.float32)
    blk = w1_ref[...]
    w1s_ref[...] = blk * (bng_ref[...] * C_BN * valid)
    bacc[...] += jnp.sum(blk * (bnb_ref[...] * valid), axis=0, keepdims=True)

    @pl.when(k == pl.num_programs(0) - 1)
    def _():
        bias_ref[...] = bacc[...]


def _tc_w(w1a_part, bng_pad, bnb_pad):
    return pl.pallas_call(
        _tc_w_body,
        grid=(16,),
        in_specs=[
            pl.BlockSpec((512, 512), lambda k: (k, 0)),
            pl.BlockSpec((512, 1), lambda k: (k, 0)),
            pl.BlockSpec((512, 1), lambda k: (k, 0)),
        ],
        out_specs=[
            pl.BlockSpec((512, 512), lambda k: (k, 0)),
            pl.BlockSpec((1, 512), lambda k: (0, 0)),
        ],
        out_shape=[
            jax.ShapeDtypeStruct((8192, 512), jnp.float32),
            jax.ShapeDtypeStruct((1, 512), jnp.float32),
        ],
        scratch_shapes=[pltpu.VMEM((1, 512), jnp.float32)],
    )(w1a_part, bng_pad, bnb_pad)


_NBUF = 4
_NBATCH = 300


def _sc_scatter_body(ev_r, ytab, out,
                     ev, r0, r1, r2, r3, i0, i1, i2, i3, d0, d1, d2, d3,
                     zbuf, acc, sem0, sem1, sem2, sem3):
    rows = (r0, r1, r2, r3)
    idxs = (i0, i1, i2, i3)
    dsts = (d0, d1, d2, d3)
    sems = (sem0, sem1, sem2, sem3)

    c = lax.axis_index("c")
    s = lax.axis_index("s")

    zero16f = jnp.zeros((16,), jnp.float32)

    def _zerozb(j, _):
        zbuf[j, :] = zero16f
        return ()
    lax.fori_loop(0, 600, _zerozb, (), unroll=8)

    pltpu.sync_copy(ev_r.at[s], ev)

    for r in range(4):
        sl = 2 * r + c

        for q in range(4):
            pltpu.sync_copy(zbuf, acc.at[pl.ds(s * 2400 + q * 600, 600)])
        plsc.subcore_barrier()

        def _start(b, q):
            for k in range(8):
                v = ev[b, pl.ds(k * 16, 16)]
                src = lax.shift_right_logical(v, 16)
                idxs[q][pl.ds(k * 16, 16)] = src * 8 + sl
            pltpu.async_copy(ytab.at[idxs[q]], rows[q], sems[q])

        for p in range(_NBUF):
            _start(p, p)

        def _group(g, _):
            for q in range(_NBUF):
                b = g * _NBUF + q
                for k in range(8):
                    v = ev[b, pl.ds(k * 16, 16)]
                    dsts[q][pl.ds(k * 16, 16)] = v & 0xFFFF
                pltpu.make_async_copy(
                    ytab.at[idxs[q]], rows[q], sems[q]).wait()
                pltpu.sync_copy(rows[q], acc.at[dsts[q]], add=True)
                nb = b + _NBUF

                @pl.when(nb < _NBATCH)
                def _():
                    _start(nb, q)
            return ()
        lax.fori_loop(0, _NBATCH // _NBUF, _group, ())

        plsc.subcore_barrier()

        pltpu.sync_copy(acc.at[pl.ds(s * 2400, 2400)],
                        out.at[pl.ds(s * 2400, 2400), pl.ds(sl * 16, 16)])

        plsc.subcore_barrier()


def _sc_scatter(ev_r, ytab):
    return pl.kernel(
        _sc_scatter_body,
        out_type=jax.ShapeDtypeStruct((N, F), jnp.float32),
        mesh=_sc_mesh(),
        compiler_params=pltpu.CompilerParams(use_tc_tiling_on_sc=False),
        scratch_types=[
            pltpu.VMEM((_NBATCH, 128), jnp.int32),
            pltpu.VMEM((128, 16), jnp.float32),
            pltpu.VMEM((128, 16), jnp.float32),
            pltpu.VMEM((128, 16), jnp.float32),
            pltpu.VMEM((128, 16), jnp.float32),
            pltpu.VMEM((128,), jnp.int32),
            pltpu.VMEM((128,), jnp.int32),
            pltpu.VMEM((128,), jnp.int32),
            pltpu.VMEM((128,), jnp.int32),
            pltpu.VMEM((128,), jnp.int32),
            pltpu.VMEM((128,), jnp.int32),
            pltpu.VMEM((128,), jnp.int32),
            pltpu.VMEM((128,), jnp.int32),
            pltpu.VMEM((600, 16), jnp.float32),
            pltpu.VMEM_SHARED((N, 16), jnp.float32),
            pltpu.SemaphoreType.DMA,
            pltpu.SemaphoreType.DMA,
            pltpu.SemaphoreType.DMA,
            pltpu.SemaphoreType.DMA,
        ],
    )(ev_r, ytab)


def _tc_a_body(x_ref, w_ref, hist_ref, y_ref):
    deg = jnp.sum(hist_ref[...].astype(jnp.float32), axis=0) + 1.0
    dinv = lax.rsqrt(deg)
    xw = jnp.dot(x_ref[...], w_ref[...], preferred_element_type=jnp.float32)
    y_ref[...] = xw * dinv[:, None]


def _tc_a(x, w_gcn, hist):
    bn = 1280
    return pl.pallas_call(
        _tc_a_body,
        grid=(N // bn,),
        in_specs=[
            pl.BlockSpec((bn, F), lambda i: (i, 0)),
            pl.BlockSpec((F, F), lambda i: (0, 0)),
            pl.BlockSpec((NW, bn), lambda i: (0, i)),
        ],
        out_specs=pl.BlockSpec((bn, F), lambda i: (i, 0)),
        out_shape=jax.ShapeDtypeStruct((N, F), jnp.float32),
    )(x, w_gcn, hist)


_GB = 10


def _tc_b_body(s_ref, y_ref, hist_ref,
               bg_ref, wa_ref, ba_ref, hm_ref):
    h = s_ref[...]
    deg = jnp.sum(hist_ref[...].astype(jnp.float32), axis=0) + 1.0
    dinv = lax.rsqrt(deg)
    h1 = jnp.tanh((h + y_ref[...]) * dinv[:, None] + bg_ref[...])
    logits = jnp.dot(h1, wa_ref[...], preferred_element_type=jnp.float32)
    logits = logits + ba_ref[...]
    m = jnp.max(logits, axis=1, keepdims=True)
    ex = jnp.exp(logits - m)
    aw = ex / jnp.sum(ex, axis=1, keepdims=True)
    ga = h1 * aw
    hm_ref[...] = jnp.mean(ga.reshape(_GB, F, F), axis=1)[:, None, :]


def _tc_b(s, y, hist, b_gcn2, wa, ba2):
    return pl.pallas_call(
        _tc_b_body,
        grid=(G // _GB,),
        in_specs=[
            pl.BlockSpec((_GB * F, F), lambda i: (i, 0)),
            pl.BlockSpec((_GB * F, F), lambda i: (i, 0)),
            pl.BlockSpec((NW, _GB * F), lambda i: (0, i)),
            pl.BlockSpec((1, F), lambda i: (0, 0)),
            pl.BlockSpec((F, F), lambda i: (0, 0)),
            pl.BlockSpec((1, F), lambda i: (0, 0)),
        ],
        out_specs=pl.BlockSpec((_GB, 1, F), lambda i: (i, 0, 0)),
        out_shape=jax.ShapeDtypeStruct((G, 1, F), jnp.float32),
    )(s, y, hist, b_gcn2, wa, ba2)


def _tc_c_body(hm_ref, wg_ref, bg_ref, wat_ref, bat_ref, bnhg_ref, bnhb_ref,
               out_ref):
    y2 = jax.nn.relu(
        jnp.dot(hm_ref[...], wg_ref[...], preferred_element_type=jnp.float32)
        + bg_ref[...])
    l2 = jnp.dot(y2, wat_ref[...], preferred_element_type=jnp.float32)
    l2 = l2 + bat_ref[...]
    l2c = l2[:, 0:1]
    m = jnp.max(l2c, axis=0, keepdims=True)
    ex = jnp.exp(l2c - m)
    aw2 = ex / jnp.sum(ex, axis=0, keepdims=True)
    hg = jnp.sum(y2 * aw2, axis=0)
    out_ref[...] = (hg * C_BN * bnhg_ref[0, :] + bnhb_ref[0, :])[None, :]


def _tc_c(hm, wg, bg2, wat_pad, bat_pad, bnhg2, bnhb2):
    full = lambda shape: pl.BlockSpec(shape, lambda: tuple(0 for _ in shape))
    return pl.pallas_call(
        _tc_c_body,
        in_specs=[full((G, F)), full((F, F)), full((1, F)), full((F, F)),
                  full((1, F)), full((1, F)), full((1, F))],
        out_specs=full((1, F)),
        out_shape=jax.ShapeDtypeStruct((1, F), jnp.float32),
    )(hm, wg, bg2, wat_pad, bat_pad, bnhg2, bnhb2)


def _tc_d_body(xp_ref, w1f_ref, w1b_ref, hgbn_ref, bias1_ref,
               b1_ref, g1_ref, e1_ref, w2_ref, b2_ref, g2_ref, e2_ref,
               w3_ref, b3_ref, g3_ref, e3_ref, w4_ref, b4_ref,
               out_ref, acc):
    i = pl.program_id(0)

    @pl.when(i == 0)
    def _():
        acc[...] = jnp.zeros_like(acc)
        out_ref[...] = jnp.zeros_like(out_ref)

    acc[...] += jnp.dot(xp_ref[...], w1f_ref[...],
                        preferred_element_type=jnp.float32)

    @pl.when(i == pl.num_programs(0) - 1)
    def _():
        hg_term = jnp.dot(hgbn_ref[...], w1b_ref[...],
                          preferred_element_type=jnp.float32)
        z1 = acc[...] + bias1_ref[...] + hg_term + b1_ref[...]
        z1 = jax.nn.relu(z1 * (C_BN * g1_ref[...]) + e1_ref[...])
        z2 = jnp.dot(z1, w2_ref[...], preferred_element_type=jnp.float32)
        z2 = jax.nn.relu((z2 + b2_ref[...]) * (C_BN * g2_ref[...])
                         + e2_ref[...])
        z3 = jnp.dot(z2, w3_ref[...], preferred_element_type=jnp.float32)
        z3 = jax.nn.relu((z3 + b3_ref[...]) * (C_BN * g3_ref[...])
                         + e3_ref[...])
        z4 = jnp.dot(z3, w4_ref[...], preferred_element_type=jnp.float32)
        out_ref[...] = z4 + b4_ref[...]


def _tc_d(xp, w1full, w1b, hgbn, bias1, b1, g1, e1, w2, b2, g2, e2,
          w3, b3, g3, e3, w4p, b4p):
    const = lambda shape: pl.BlockSpec(shape, lambda i: tuple(0 for _ in shape))
    return pl.pallas_call(
        _tc_d_body,
        grid=(F * F // 512,),
        in_specs=[
            pl.BlockSpec((G, 512), lambda i: (0, i)),
            pl.BlockSpec((512, 512), lambda i: (i, 0)),
            const((F, 512)),
            const((1, F)),
            const((1, 512)),
            const((1, 512)), const((1, 512)), const((1, 512)),
            const((512, 256)),
            const((1, 256)), const((1, 256)), const((1, 256)),
            const((256, 256)),
            const((1, 256)), const((1, 256)), const((1, 256)),
            const((256, F)),
            const((1, F)),
        ],
        out_specs=const((G, F)),
        out_shape=jax.ShapeDtypeStruct((G, F), jnp.float32),
        scratch_shapes=[
            pltpu.VMEM((G, 512), jnp.float32),
        ],
    )(xp, w1full, w1b, hgbn, bias1, b1, g1, e1, w2, b2, g2, e2,
      w3, b3, g3, e3, w4p, b4p)


def kernel(x, edge_index, batch, W_gcn, b_gcn, Wa, ba, Wg, bg, Wat, bat,
           bn_g, bn_b, bnh_g, bnh_b, W1, b1, g1, e1, W2, b2, g2, e2,
           W3, b3, g3, e3, W4, b4):
    dst_r2 = edge_index[1].reshape(NW, 150, 128)
    ev_r = ((edge_index[0] << 16) | edge_index[1]).reshape(NS, _NBATCH, 128)

    w1idx = jnp.asarray(_W1IDX)
    w1s, bias1 = _tc_w(W1[:8192],
                       jnp.pad(bn_g, (0, 64)).reshape(8192, 1),
                       jnp.pad(bn_b, (0, 64)).reshape(8192, 1))

    hist = _sc_hist(dst_r2)
    w1full = _sc_w1x(w1s, w1idx)

    y = _tc_a(x, W_gcn, hist)
    ytab = y.reshape(N * 8, 16)

    s = _sc_scatter(ev_r, ytab)

    hm = _tc_b(s, y, hist, b_gcn.reshape(1, F), Wa, ba.reshape(1, F))
    hm2 = hm.reshape(G, F)

    hgbn = _tc_c(hm2, Wg, bg.reshape(1, F),
                 jnp.pad(Wat, ((0, 0), (0, F - 1))),
                 jnp.pad(bat, (0, F - 1)).reshape(1, F),
                 bnh_g.reshape(1, F), bnh_b.reshape(1, F))

    xp = x.reshape(G, F * F)
    zd = _tc_d(xp, w1full, W1[D_TRI:], hgbn, bias1,
               b1.reshape(1, 512), g1.reshape(1, 512), e1.reshape(1, 512),
               W2, b2.reshape(1, 256), g2.reshape(1, 256), e2.reshape(1, 256),
               W3, b3.reshape(1, 256), g3.reshape(1, 256), e3.reshape(1, 256),
               jnp.pad(W4, ((0, 0), (0, F - 2))),
               jnp.pad(b4, (0, F - 2)).reshape(1, F))
    return zd[:, :2]
